# baseline (speedup 1.0000x reference)
import functools

import jax
import jax.numpy as jnp
from jax import lax
from jax.experimental import pallas as pl
from jax.experimental.pallas import tpu as pltpu
from jax.experimental.pallas import tpu_sc as plsc

_NBUF = 5


def _emb_kernel(S, D, V, B, NC, NW):
    BLK = B // NW
    mesh = plsc.VectorSubcoreMesh(core_axis_name="c", subcore_axis_name="s")
    n_iter = S // _NBUF

    @functools.partial(
        pl.kernel,
        mesh=mesh,
        out_type=jax.ShapeDtypeStruct((S, D, B), jnp.float32),
        scratch_types=[
            pltpu.VMEM((V * D,), jnp.float32),
            pltpu.VMEM((S, BLK), jnp.int32),
        ] + [pltpu.VMEM((D, BLK), jnp.float32) for _ in range(_NBUF)]
          + [pltpu.SemaphoreType.DMA for _ in range(_NBUF + 1)],
        compiler_params=pltpu.CompilerParams(needs_layout_passes=False),
    )
    def k(wt_hbm, xt_hbm, out_hbm, wt_v, idx_all, *rest):
        bufs = rest[:_NBUF]
        wsems = rest[_NBUF:2 * _NBUF]
        isem = rest[2 * _NBUF]
        wid = lax.axis_index("s") * NC + lax.axis_index("c")
        col0 = wid * BLK

        pltpu.async_copy(xt_hbm.at[:, pl.ds(col0, BLK)], idx_all, isem)
        pltpu.sync_copy(wt_hbm, wt_v)
        pltpu.make_async_copy(
            xt_hbm.at[:, pl.ds(col0, BLK)], idx_all, isem).wait()

        def body(i, carry):
            for h in range(_NBUF):
                s = _NBUF * i + h
                buf = bufs[h]
                cvecs = [idx_all[s, pl.ds(g * 16, 16)]
                         for g in range(BLK // 16)]

                @pl.when(i > 0)
                def _():
                    pltpu.make_async_copy(
                        buf, out_hbm.at[s, :, pl.ds(col0, BLK)],
                        wsems[h]).wait()

                blocks = [(g, d0) for g in range(BLK // 16)
                          for d0 in range(0, D, 16)]
                prev = None
                for g, d0 in blocks:
                    cvec = cvecs[g]
                    cur = []
                    for u in range(16):
                        cur.append(
                            plsc.load_gather(wt_v, [cvec + (d0 + u) * V]))
                        if prev is not None:
                            pg, pd0, pvals = prev
                            buf[pd0 + u, pl.ds(pg * 16, 16)] = pvals[u]
                    prev = (g, d0, cur)
                pg, pd0, pvals = prev
                for u in range(16):
                    buf[pd0 + u, pl.ds(pg * 16, 16)] = pvals[u]

                pltpu.async_copy(
                    buf, out_hbm.at[s, :, pl.ds(col0, BLK)], wsems[h])
            return carry

        lax.fori_loop(0, n_iter, body, 0)
        for h in range(_NBUF):
            s = S - _NBUF + h
            pltpu.make_async_copy(
                bufs[h], out_hbm.at[s, :, pl.ds(col0, BLK)], wsems[h]).wait()

    return k


def kernel(x, weights):
    Bdim, S = x.shape
    V, D = weights.shape
    info = plsc.get_sparse_core_info()
    NC, NS = info.num_cores, info.num_subcores
    NW = NC * NS
    wt_flat = weights.astype(jnp.float32).T.reshape(V * D)
    xt = x.astype(jnp.int32).T
    k = _emb_kernel(S, D, V, Bdim, NC, NW)
    out_t = k(wt_flat, xt)
    return jnp.transpose(out_t, (2, 0, 1))
